# baseline (speedup 1.0000x reference)
import functools

import jax
import jax.numpy as jnp
from jax import lax
from jax.experimental import pallas as pl
from jax.experimental.pallas import tpu as pltpu
from jax.experimental.pallas import tpu_sc as plsc

N = 10000
D_IN = 256
D_HID = 128
E = 160000

NC = 2
NS = 16
NW = NC * NS
CB = 128
CHUNKS = 40
E_PAD = NW * CHUNKS * CB
NCHUNKS = E_PAD // CB
N0 = 80
NMAX = N0
NPAD = 10240
STRIPE = NPAD // NS
RB = 1000

_mesh = plsc.VectorSubcoreMesh(core_axis_name="c", subcore_axis_name="s")


@functools.partial(
    pl.kernel,
    out_type=jax.ShapeDtypeStruct((NC, NPAD), jnp.float32),
    mesh=_mesh,
    scratch_types=[
        pltpu.VMEM((CHUNKS, CB), jnp.int32),
        pltpu.VMEM((CB,), jnp.float32),
        pltpu.VMEM((STRIPE,), jnp.float32),
        pltpu.VMEM_SHARED((NPAD,), jnp.float32),
    ],
)
def _deg_kernel(dst3, deg_out, dstv, ones_v, zer_v, acc):
    cid = lax.axis_index("c")
    sid = lax.axis_index("s")
    wid = cid * NS + sid

    def _zero(i, _):
        zer_v[pl.ds(i * 16, 16)] = jnp.zeros((16,), jnp.float32)
        return 0
    lax.fori_loop(0, STRIPE // 16, _zero, 0)

    def _one(i, _):
        ones_v[pl.ds(i * 16, 16)] = jnp.ones((16,), jnp.float32)
        return 0
    lax.fori_loop(0, CB // 16, _one, 0)

    pltpu.sync_copy(zer_v, acc.at[pl.ds(sid * STRIPE, STRIPE)])
    plsc.subcore_barrier()

    pltpu.sync_copy(dst3.at[wid], dstv)

    def _chunk(c, _):
        pltpu.sync_copy(ones_v, acc.at[dstv.at[c]], add=True)
        return 0
    lax.fori_loop(0, CHUNKS, _chunk, 0)
    plsc.subcore_barrier()

    pltpu.sync_copy(acc.at[pl.ds(sid * STRIPE, STRIPE)],
                    deg_out.at[cid, pl.ds(sid * STRIPE, STRIPE)])


def _make_agg_kernel(D):
    def body(g_hbm, srcf, dstf, agg_out, srcv, dstv, b0, zer_v, acc):
        cid = lax.axis_index("c")
        sid = lax.axis_index("s")
        nv = D // 16
        pl.when(cid == 0)(lambda: _core0(g_hbm, srcf, dstf, agg_out, srcv,
                                         dstv, b0, zer_v, acc, sid))

    def _core0(g_hbm, srcf, dstf, agg_out, srcv, dstv, b0, zer_v, acc, sid):
        nv = D // 16

        def _zero(k, _):
            i = k // nv
            j = k % nv
            zer_v[i, pl.ds(j * 16, 16)] = jnp.zeros((16,), jnp.float32)
            return 0
        lax.fori_loop(0, 64 * nv, _zero, 0)

        def _zstripe(i, _):
            pltpu.sync_copy(zer_v, acc.at[pl.ds(sid * STRIPE + i * 64, 64)])
            return 0
        lax.fori_loop(0, STRIPE // 64, _zstripe, 0)
        plsc.subcore_barrier()

        start = sid * N0
        pltpu.sync_copy(srcf.at[pl.ds(start, N0)], srcv)
        pltpu.sync_copy(dstf.at[pl.ds(start, N0)], dstv)

        def _chunk(c, _):
            pltpu.sync_copy(g_hbm.at[srcv.at[c]], b0)
            pltpu.sync_copy(b0, acc.at[dstv.at[c]], add=True)
            return 0
        lax.fori_loop(0, N0, _chunk, 0)
        plsc.subcore_barrier()

        pltpu.sync_copy(acc.at[pl.ds(sid * STRIPE, STRIPE)],
                        agg_out.at[pl.ds(sid * STRIPE, STRIPE)])

    return pl.kernel(
        body,
        out_type=jax.ShapeDtypeStruct((NPAD, D), jnp.float32),
        mesh=_mesh,
        scratch_types=[
            pltpu.VMEM((NMAX, CB), jnp.int32),
            pltpu.VMEM((NMAX, CB), jnp.int32),
            pltpu.VMEM((CB, D), jnp.float32),
            pltpu.VMEM((64, D), jnp.float32),
            pltpu.VMEM_SHARED((NPAD, D), jnp.float32),
        ],
    )


_agg_kernel = _make_agg_kernel(D_HID)


def _t1_body(x_ref, w_ref, deg_ref, g_ref):
    dinv = lax.rsqrt(deg_ref[0] + deg_ref[1] + 1.0)
    g_ref[...] = dinv * jnp.dot(x_ref[...], w_ref[...],
                                preferred_element_type=jnp.float32)


def _t2_body(a_ref, g_ref, b_ref, w_ref, deg_ref, o_ref):
    dinv = lax.rsqrt(deg_ref[0] + deg_ref[1] + 1.0)
    h = dinv * (a_ref[...] + g_ref[...]) + b_ref[...]
    h = jnp.maximum(h, 0.0)
    o_ref[...] = dinv * jnp.dot(h, w_ref[...],
                                preferred_element_type=jnp.float32)


def _t3_body(a_ref, g_ref, b_ref, deg_ref, o_ref):
    dinv = lax.rsqrt(deg_ref[0] + deg_ref[1] + 1.0)
    h = dinv * (a_ref[...] + g_ref[...]) + b_ref[...]
    o_ref[...] = dinv * jnp.maximum(h, 0.0)


def _t4_body(a_ref, u_ref, w_ref, b_ref, deg_ref, o_ref):
    dinv = lax.rsqrt(deg_ref[0] + deg_ref[1] + 1.0)
    t = dinv * (a_ref[...] + u_ref[...])
    o_ref[...] = jnp.dot(t, w_ref[...],
                         preferred_element_type=jnp.float32) + b_ref[...]


def _row_spec(width):
    return pl.BlockSpec((RB, width), lambda i: (i, 0))


def _full(shape):
    return pl.BlockSpec(shape, lambda i: tuple(0 for _ in shape))


_deg_spec = pl.BlockSpec((NC, RB, 1), lambda i: (0, i, 0))
_agg_spec = pl.BlockSpec((RB, D_HID), lambda i: (i, 0))
_grid = (N // RB,)


def kernel(x, edge_index, W1, b1, W2, b2, W3, b3):
    pad = E_PAD - E
    srcf = jnp.concatenate(
        [edge_index[0].astype(jnp.int32), jnp.zeros((pad,), jnp.int32)]
    ).reshape(NCHUNKS, CB)
    dstf = jnp.concatenate(
        [edge_index[1].astype(jnp.int32), jnp.full((pad,), N, jnp.int32)]
    ).reshape(NCHUNKS, CB)

    degp = _deg_kernel(dstf.reshape(NW, CHUNKS, CB)).reshape(NC, NPAD, 1)

    g1 = pl.pallas_call(
        _t1_body,
        grid=_grid,
        in_specs=[_row_spec(D_IN), _full((D_IN, D_HID)), _deg_spec],
        out_specs=_row_spec(D_HID),
        out_shape=jax.ShapeDtypeStruct((N, D_HID), jnp.float32),
    )(x, W1, degp)

    a1 = _agg_kernel(g1, srcf, dstf)

    g2 = pl.pallas_call(
        _t2_body,
        grid=_grid,
        in_specs=[_agg_spec, _row_spec(D_HID), _full((1, D_HID)),
                  _full((D_HID, D_HID)), _deg_spec],
        out_specs=_row_spec(D_HID),
        out_shape=jax.ShapeDtypeStruct((N, D_HID), jnp.float32),
    )(a1, g1, b1.reshape(1, D_HID), W2, degp)

    a2 = _agg_kernel(g2, srcf, dstf)

    u3 = pl.pallas_call(
        _t3_body,
        grid=_grid,
        in_specs=[_agg_spec, _row_spec(D_HID), _full((1, D_HID)), _deg_spec],
        out_specs=_row_spec(D_HID),
        out_shape=jax.ShapeDtypeStruct((N, D_HID), jnp.float32),
    )(a2, g2, b2.reshape(1, D_HID), degp)

    a3 = _agg_kernel(u3, srcf, dstf)

    W3p = jnp.pad(W3, ((0, 0), (0, 8 - W3.shape[1])))
    b3p = jnp.pad(b3, (0, 8 - b3.shape[0])).reshape(1, 8)
    outp = pl.pallas_call(
        _t4_body,
        grid=_grid,
        in_specs=[_agg_spec, _row_spec(D_HID), _full((D_HID, 8)),
                  _full((1, 8)), _deg_spec],
        out_specs=_row_spec(8),
        out_shape=jax.ShapeDtypeStruct((N, 8), jnp.float32),
    )(a3, u3, W3p, b3p, degp)

    return outp[:, : W3.shape[1]]


# speedup vs baseline: 1.5540x; 1.5540x over previous
import functools

import jax
import jax.numpy as jnp
from jax import lax
from jax.experimental import pallas as pl
from jax.experimental.pallas import tpu as pltpu
from jax.experimental.pallas import tpu_sc as plsc

N = 10000
D_IN = 256
D_HID = 128
E = 160000

NC = 2
NS = 16
NW = NC * NS
CB = 128
CHUNKS = 40
E_PAD = NW * CHUNKS * CB
NCHUNKS = E_PAD // CB
N0 = 72
N1 = 80 - N0
NMAX = max(N0, N1)
NPAD = 10240
STRIPE = NPAD // NS
RB = 2000

_mesh = plsc.VectorSubcoreMesh(core_axis_name="c", subcore_axis_name="s")


@functools.partial(
    pl.kernel,
    out_type=jax.ShapeDtypeStruct((NC, NPAD), jnp.float32),
    mesh=_mesh,
    scratch_types=[
        pltpu.VMEM((CHUNKS, CB), jnp.int32),
        pltpu.VMEM((CB,), jnp.float32),
        pltpu.VMEM((STRIPE,), jnp.float32),
        pltpu.VMEM_SHARED((NPAD,), jnp.float32),
    ],
)
def _deg_kernel(dst3, deg_out, dstv, ones_v, zer_v, acc):
    cid = lax.axis_index("c")
    sid = lax.axis_index("s")
    wid = cid * NS + sid

    def _zero(i, _):
        zer_v[pl.ds(i * 16, 16)] = jnp.zeros((16,), jnp.float32)
        return 0
    lax.fori_loop(0, STRIPE // 16, _zero, 0)

    def _one(i, _):
        ones_v[pl.ds(i * 16, 16)] = jnp.ones((16,), jnp.float32)
        return 0
    lax.fori_loop(0, CB // 16, _one, 0)

    pltpu.sync_copy(zer_v, acc.at[pl.ds(sid * STRIPE, STRIPE)])
    plsc.subcore_barrier()

    pltpu.sync_copy(dst3.at[wid], dstv)

    def _chunk(c, _):
        pltpu.sync_copy(ones_v, acc.at[dstv.at[c]], add=True)
        return 0
    lax.fori_loop(0, CHUNKS, _chunk, 0)
    plsc.subcore_barrier()

    pltpu.sync_copy(acc.at[pl.ds(sid * STRIPE, STRIPE)],
                    deg_out.at[cid, pl.ds(sid * STRIPE, STRIPE)])


def _make_agg_kernel(D):
    def body(g_hbm, srcf, dstf, agg_out, srcv, dstv, b0, zer_v, acc):
        cid = lax.axis_index("c")
        sid = lax.axis_index("s")
        nv = D // 16

        def _zero(k, _):
            i = k // nv
            j = k % nv
            zer_v[i, pl.ds(j * 16, 16)] = jnp.zeros((16,), jnp.float32)
            return 0
        lax.fori_loop(0, 64 * nv, _zero, 0)

        def _zstripe(i, _):
            pltpu.sync_copy(zer_v, acc.at[pl.ds(sid * STRIPE + i * 64, 64)])
            return 0
        lax.fori_loop(0, STRIPE // 64, _zstripe, 0)
        plsc.subcore_barrier()

        def _run(start, n):
            pltpu.sync_copy(srcf.at[pl.ds(start, n)], srcv.at[pl.ds(0, n)])
            pltpu.sync_copy(dstf.at[pl.ds(start, n)], dstv.at[pl.ds(0, n)])

            def _chunk(c, _):
                pltpu.sync_copy(g_hbm.at[srcv.at[c]], b0)
                pltpu.sync_copy(b0, acc.at[dstv.at[c]], add=True)
                return 0
            lax.fori_loop(0, n, _chunk, 0)

        @pl.when(cid == 0)
        def _():
            _run(sid * N0, N0)

        @pl.when(cid == 1)
        def _():
            _run(NS * N0 + sid * N1, N1)
        plsc.subcore_barrier()

        pltpu.sync_copy(acc.at[pl.ds(sid * STRIPE, STRIPE)],
                        agg_out.at[cid, pl.ds(sid * STRIPE, STRIPE)])

    return pl.kernel(
        body,
        out_type=jax.ShapeDtypeStruct((NC, NPAD, D), jnp.float32),
        mesh=_mesh,
        scratch_types=[
            pltpu.VMEM((NMAX, CB), jnp.int32),
            pltpu.VMEM((NMAX, CB), jnp.int32),
            pltpu.VMEM((CB, D), jnp.float32),
            pltpu.VMEM((64, D), jnp.float32),
            pltpu.VMEM_SHARED((NPAD, D), jnp.float32),
        ],
    )


_agg_kernel = _make_agg_kernel(D_HID)


def _t1_body(x_ref, w_ref, deg_ref, g_ref):
    dinv = lax.rsqrt(deg_ref[0] + deg_ref[1] + 1.0)
    g_ref[...] = dinv * jnp.dot(x_ref[...], w_ref[...],
                                preferred_element_type=jnp.float32)


def _t2_body(a_ref, g_ref, b_ref, w_ref, deg_ref, o_ref):
    dinv = lax.rsqrt(deg_ref[0] + deg_ref[1] + 1.0)
    h = dinv * (a_ref[0] + a_ref[1] + g_ref[...]) + b_ref[...]
    h = jnp.maximum(h, 0.0)
    o_ref[...] = dinv * jnp.dot(h, w_ref[...],
                                preferred_element_type=jnp.float32)


def _t3_body(a_ref, g_ref, b_ref, deg_ref, o_ref):
    dinv = lax.rsqrt(deg_ref[0] + deg_ref[1] + 1.0)
    h = dinv * (a_ref[0] + a_ref[1] + g_ref[...]) + b_ref[...]
    o_ref[...] = dinv * jnp.maximum(h, 0.0)


def _t4_body(a_ref, u_ref, w_ref, b_ref, deg_ref, o_ref):
    dinv = lax.rsqrt(deg_ref[0] + deg_ref[1] + 1.0)
    t = dinv * (a_ref[0] + a_ref[1] + u_ref[...])
    o_ref[...] = jnp.dot(t, w_ref[...],
                         preferred_element_type=jnp.float32) + b_ref[...]


def _row_spec(width):
    return pl.BlockSpec((RB, width), lambda i: (i, 0))


def _full(shape):
    return pl.BlockSpec(shape, lambda i: tuple(0 for _ in shape))


_deg_spec = pl.BlockSpec((NC, RB, 1), lambda i: (0, i, 0))
_agg_spec = pl.BlockSpec((NC, RB, D_HID), lambda i: (0, i, 0))
_grid = (N // RB,)


def kernel(x, edge_index, W1, b1, W2, b2, W3, b3):
    pad = E_PAD - E
    srcf = jnp.concatenate(
        [edge_index[0].astype(jnp.int32), jnp.zeros((pad,), jnp.int32)]
    ).reshape(NCHUNKS, CB)
    dstf = jnp.concatenate(
        [edge_index[1].astype(jnp.int32), jnp.full((pad,), N, jnp.int32)]
    ).reshape(NCHUNKS, CB)

    degp = _deg_kernel(dstf.reshape(NW, CHUNKS, CB)).reshape(NC, NPAD, 1)

    g1 = pl.pallas_call(
        _t1_body,
        grid=_grid,
        in_specs=[_row_spec(D_IN), _full((D_IN, D_HID)), _deg_spec],
        out_specs=_row_spec(D_HID),
        out_shape=jax.ShapeDtypeStruct((N, D_HID), jnp.float32),
    )(x, W1, degp)

    a1 = _agg_kernel(g1, srcf, dstf)

    g2 = pl.pallas_call(
        _t2_body,
        grid=_grid,
        in_specs=[_agg_spec, _row_spec(D_HID), _full((1, D_HID)),
                  _full((D_HID, D_HID)), _deg_spec],
        out_specs=_row_spec(D_HID),
        out_shape=jax.ShapeDtypeStruct((N, D_HID), jnp.float32),
    )(a1, g1, b1.reshape(1, D_HID), W2, degp)

    a2 = _agg_kernel(g2, srcf, dstf)

    u3 = pl.pallas_call(
        _t3_body,
        grid=_grid,
        in_specs=[_agg_spec, _row_spec(D_HID), _full((1, D_HID)), _deg_spec],
        out_specs=_row_spec(D_HID),
        out_shape=jax.ShapeDtypeStruct((N, D_HID), jnp.float32),
    )(a2, g2, b2.reshape(1, D_HID), degp)

    a3 = _agg_kernel(u3, srcf, dstf)

    W3p = jnp.pad(W3, ((0, 0), (0, 8 - W3.shape[1])))
    b3p = jnp.pad(b3, (0, 8 - b3.shape[0])).reshape(1, 8)
    outp = pl.pallas_call(
        _t4_body,
        grid=_grid,
        in_specs=[_agg_spec, _row_spec(D_HID), _full((D_HID, 8)),
                  _full((1, 8)), _deg_spec],
        out_specs=_row_spec(8),
        out_shape=jax.ShapeDtypeStruct((N, 8), jnp.float32),
    )(a3, u3, W3p, b3p, degp)

    return outp[:, : W3.shape[1]]
